# baseline (speedup 1.0000x reference)
import functools
import math

import jax
import jax.numpy as jnp
from jax import lax
from jax.experimental import pallas as pl
from jax.experimental.pallas import tpu as pltpu
from jax.experimental.pallas import tpu_sc as plsc

NUM_EMB = 1000000
DIM = 64
MAX_LEN = 5000
BATCH = 4096
SEQ = 200

ROWS = BATCH * SEQ
CHUNK = 128
NCHUNKS = ROWS // CHUNK
PE_EXT = SEQ + CHUNK


def _pos_embedding_ext():
    position = jnp.arange(0, SEQ, dtype=jnp.float32)[:, None]
    div_term = jnp.arange(0, DIM, 2, dtype=jnp.float32)
    div_term = jnp.exp(div_term * (-math.log(10000.0) / DIM))
    pe = jnp.zeros((SEQ, DIM), dtype=jnp.float32)
    pe = pe.at[:, 0::2].set(jnp.sin(position * div_term))
    pe = pe.at[:, 1::2].set(jnp.cos(position * div_term))
    return jnp.concatenate([pe, pe[:CHUNK]], axis=0)


def kernel(data, table):
    info = plsc.get_sparse_core_info()
    nc, ns = info.num_cores, info.num_subcores
    nw = nc * ns
    chunks_per_w = NCHUNKS // nw
    rows_per_w = chunks_per_w * CHUNK

    idx2d = data.reshape(NCHUNKS, CHUNK).astype(jnp.int32)
    pe_ext = _pos_embedding_ext()

    mesh = plsc.VectorSubcoreMesh(core_axis_name="c", subcore_axis_name="s")

    @functools.partial(
        pl.kernel,
        mesh=mesh,
        compiler_params=pltpu.CompilerParams(use_tc_tiling_on_sc=False),
        out_type=jax.ShapeDtypeStruct((ROWS, DIM), jnp.float32),
        scratch_types=[
            pltpu.VMEM((chunks_per_w, CHUNK), jnp.int32),
            pltpu.VMEM((PE_EXT, DIM), jnp.float32),
            pltpu.VMEM((CHUNK, DIM), jnp.float32),
            pltpu.VMEM((CHUNK, DIM), jnp.float32),
            pltpu.SemaphoreType.DMA,
            pltpu.SemaphoreType.DMA,
            pltpu.SemaphoreType.DMA,
            pltpu.SemaphoreType.DMA,
        ],
    )
    def emb_kernel(idx_hbm, table_hbm, pe_hbm, out_hbm,
                   idx_v, pe_v, buf0, buf1, gsem0, gsem1, ssem0, ssem1):
        wid = lax.axis_index("s") * nc + lax.axis_index("c")
        cbase = wid * chunks_per_w
        obase = wid * rows_per_w

        pltpu.sync_copy(idx_hbm.at[pl.ds(cbase, chunks_per_w)], idx_v)
        pltpu.sync_copy(pe_hbm, pe_v)

        def start_gather(c, buf, gsem):
            pltpu.make_async_copy(table_hbm.at[idx_v.at[c]], buf, gsem).start()

        def wait_gather(c, buf, gsem):
            pltpu.make_async_copy(table_hbm.at[idx_v.at[c]], buf, gsem).wait()

        def start_store(c, buf, ssem):
            pltpu.make_async_copy(
                buf, out_hbm.at[pl.ds(obase + c * CHUNK, CHUNK)], ssem).start()

        def wait_store(c, buf, ssem):
            pltpu.make_async_copy(
                buf, out_hbm.at[pl.ds(obase + c * CHUNK, CHUNK)], ssem).wait()

        def add_pe(buf, p0):
            def body(i, carry):
                s = p0 + i
                for j in range(4):
                    sl = pl.ds(j * 16, 16)
                    plsc.addupdate(buf.at[i, sl], pe_v[s, sl])
                return carry
            lax.fori_loop(0, CHUNK, body, 0, unroll=8)

        def step(c, buf, gsem, ssem, nbuf, ngsem, nssem):
            @pl.when(c + 1 < chunks_per_w)
            def _():
                @pl.when(c >= 1)
                def _():
                    wait_store(c - 1, nbuf, nssem)
                start_gather(c + 1, nbuf, ngsem)

            wait_gather(c, buf, gsem)
            add_pe(buf, lax.rem(c * CHUNK, SEQ))
            start_store(c, buf, ssem)

        start_gather(0, buf0, gsem0)

        def outer(g, carry):
            step(2 * g, buf0, gsem0, ssem0, buf1, gsem1, ssem1)
            step(2 * g + 1, buf1, gsem1, ssem1, buf0, gsem0, ssem0)
            return carry

        lax.fori_loop(0, chunks_per_w // 2, outer, 0)

        wait_store(chunks_per_w - 2, buf0, ssem0)
        wait_store(chunks_per_w - 1, buf1, ssem1)

    out = emb_kernel(idx2d, table, pe_ext)
    return out.reshape(BATCH, SEQ, DIM)
